# baseline (speedup 1.0000x reference)
import jax
import jax.numpy as jnp
from jax import lax
from jax.experimental import pallas as pl
from jax.experimental.pallas import tpu as pltpu

B, HW, C, K = 16, 196, 384, 12
REL = 48
H = W = 14
N = B * HW
PAD = 144
ROWS = PAD + HW + PAD
F32 = jnp.float32


def _dot(a, b, dims):
    return lax.dot_general(a, b, (dims, ((), ())), preferred_element_type=F32)


def _s1_body(x_ref, w0, w01, w02, w11, w12, w21, w22, cb_ref, W3_ref,
             xs_ref, xh_ref, pad_ref, s0_ref, t0_ref, t1_ref):
    b = pl.program_id(0)
    wio = lax.broadcasted_iota(jnp.int32, (HW, 1), 0) % W

    @pl.when(b == 0)
    def _():
        pad_ref[...] = jnp.zeros((ROWS, C), F32)

    def put(img):
        pad_ref[pl.ds(PAD, HW), :] = img

    def conv(wt_ref, kh, kw, ph, pw):
        acc = jnp.zeros((HW, C), F32)
        t = 0
        for i in range(kh):
            for j in range(kw):
                delta = (i - ph) * W + (j - pw)
                sl = pad_ref[pl.ds(PAD + delta, HW), :]
                dw = j - pw
                if dw != 0:
                    m = (wio + dw >= 0) & (wio + dw < W)
                    sl = jnp.where(m, sl, 0.0)
                acc = acc + sl * wt_ref[pl.ds(t, 1), :]
                t += 1
        return acc

    def brow(i):
        return cb_ref[pl.ds(i, 1), :]

    X = x_ref[0]
    put(X)
    s0 = conv(w0, 5, 5, 2, 2) + brow(0)
    s0_ref[...] = s0
    put(s0)
    t0_ref[...] = conv(w01, 1, 7, 0, 3) + brow(1)
    t1_ref[...] = conv(w11, 1, 11, 0, 5) + brow(3)
    aw2 = conv(w21, 1, 21, 0, 10) + brow(5)
    put(aw2)
    acc = s0_ref[...] + conv(w22, 21, 1, 10, 0) + brow(6)
    put(t0_ref[...])
    acc = acc + conv(w02, 7, 1, 3, 0) + brow(2)
    put(t1_ref[...])
    acc = acc + conv(w12, 11, 1, 5, 0) + brow(4)
    attn3 = _dot(acc, W3_ref[...], ((1,), (1,))) + brow(7)
    xs_ref[0] = attn3 * X
    xh = jnp.zeros((W, C), F32)
    for t in range(H):
        xh = xh + X[t * W:(t + 1) * W, :]
    xh_ref[0] = xh


def _stage1(x, taps, cbias, W3):
    w0, w01, w02, w11, w12, w21, w22 = taps
    full = lambda s: pl.BlockSpec(s, lambda b: tuple(0 for _ in s))
    return pl.pallas_call(
        _s1_body,
        grid=(B,),
        in_specs=[pl.BlockSpec((1, HW, C), lambda b: (b, 0, 0))]
        + [full(w.shape) for w in (w0, w01, w02, w11, w12, w21, w22)]
        + [full(cbias.shape), full(W3.shape)],
        out_specs=[pl.BlockSpec((1, HW, C), lambda b: (b, 0, 0)),
                   pl.BlockSpec((1, W, C), lambda b: (b, 0, 0))],
        out_shape=[jax.ShapeDtypeStruct((B, HW, C), F32),
                   jax.ShapeDtypeStruct((B, W, C), F32)],
        scratch_shapes=[pltpu.VMEM((ROWS, C), F32), pltpu.VMEM((HW, C), F32),
                        pltpu.VMEM((HW, C), F32), pltpu.VMEM((HW, C), F32)],
        compiler_params=pltpu.CompilerParams(
            dimension_semantics=("arbitrary",)),
    )(x, w0, w01, w02, w11, w12, w21, w22, cbias, W3)


def _s2_body(xs_ref, cw_ref, cb_ref, cg_ref, cbe_ref, out_ref,
             s_ref, G_ref, ab_ref):
    k = pl.program_id(0)
    b = pl.program_id(1)

    @pl.when((k == 0) & (b == 0))
    def _():
        ssum = jnp.zeros((1, C), F32)
        for bb in range(B):
            ssum = ssum + jnp.sum(xs_ref[bb], axis=0, keepdims=True)
        s_ref[...] = ssum / N
        G = jnp.zeros((C, C), F32)
        for bb in range(B):
            xc = xs_ref[bb] - s_ref[...]
            G = G + _dot(xc, xc, ((0,), (0,)))
        G_ref[...] = G

    @pl.when(b == 0)
    def _():
        Wk = cw_ref[0]
        s = s_ref[...]
        Ez = _dot(s, Wk, ((1,), (1,)))
        WG = _dot(Wk, G_ref[...], ((1,), (0,)))
        ones = jnp.ones((1, C), F32)
        var = _dot(ones, WG * Wk, ((1,), (1,))) / N
        alpha = cg_ref[0] / jnp.sqrt(var + 1e-5)
        beta = cbe_ref[0] - Ez * alpha
        ab_ref[pl.ds(0, 1), :] = alpha
        ab_ref[pl.ds(1, 1), :] = beta

    Z = _dot(xs_ref[b], cw_ref[0], ((1,), (1,)))
    y = jnp.maximum(Z * ab_ref[pl.ds(0, 1), :] + ab_ref[pl.ds(1, 1), :], 0.0)
    out_ref[...] = jnp.mean(y, axis=0, keepdims=True)[None]


def _stage2(xs, cW, cb, cg, cbe):
    full = lambda s: pl.BlockSpec(s, lambda k, b: tuple(0 for _ in s))
    out = pl.pallas_call(
        _s2_body,
        grid=(K, B),
        in_specs=[full((B, HW, C)),
                  pl.BlockSpec((1, C, C), lambda k, b: (k, 0, 0)),
                  pl.BlockSpec((1, 1, C), lambda k, b: (k, 0, 0)),
                  pl.BlockSpec((1, 1, C), lambda k, b: (k, 0, 0)),
                  pl.BlockSpec((1, 1, C), lambda k, b: (k, 0, 0))],
        out_specs=pl.BlockSpec((1, 1, C), lambda k, b: (k * B + b, 0, 0)),
        out_shape=jax.ShapeDtypeStruct((K * B, 1, C), F32),
        scratch_shapes=[pltpu.VMEM((1, C), F32), pltpu.VMEM((C, C), F32),
                        pltpu.VMEM((2, C), F32)],
        compiler_params=pltpu.CompilerParams(
            dimension_semantics=("arbitrary", "arbitrary")),
    )(xs, cW, cb, cg, cbe)
    return out.reshape(K, B, C).transpose(1, 0, 2)


def _s3_body(v_ref, xh_ref, VW_ref, Vb_ref, UW_ref, Ub_ref, gK_ref, bK_ref,
             c1w_ref, c1b_ref, c2w_ref, c2b_ref, c3w_ref, c3b_ref,
             c4w_ref, c4b_ref, cclW_ref, cclb_ref, cclg_ref, cclbe_ref,
             pL_ref, pR_ref, pb_ref, pg_ref, pbe_ref, sc_ref,
             out_ref, agg_ref, fp_ref):
    ones_row = jnp.ones((1, C), F32)

    pool_rows = []
    for b in range(B):
        XHb = xh_ref[b]
        XHm = XHb * (1.0 / W)
        x1 = _dot(XHm, c1w_ref[...], ((1,), (1,))) + c1b_ref[...]
        x2 = _dot(XHm, c2w_ref[...], ((1,), (1,))) + c2b_ref[...]
        TT = jnp.zeros((W, REL), F32)
        for u in range(W):
            TT = TT + jnp.tanh(x1[u:u + 1, :] - x2)
        P1T = _dot(TT, c4w_ref[...], ((1,), (1,))) + float(W) * c4b_ref[...]
        S3T = _dot(XHb, c3w_ref[...], ((1,), (1,))) + float(W) * c3b_ref[...]
        pool_rows.append(jnp.sum(P1T * S3T, axis=0, keepdims=True) / float(HW))
    pool = jnp.concatenate(pool_rows, axis=0)

    y1 = _dot(pool, cclW_ref[...], ((1,), (1,))) + cclb_ref[...]
    m2 = jnp.mean(y1, axis=0, keepdims=True)
    v2 = jnp.mean((y1 - m2) ** 2, axis=0, keepdims=True)
    lin = jnp.maximum(
        (y1 - m2) * lax.rsqrt(v2 + 1e-5) * cclg_ref[...] + cclbe_ref[...], 0.0)

    msum = jnp.zeros((K, 1), F32)
    iot = lax.broadcasted_iota(jnp.int32, (K, K), 1)
    for b in range(B):
        vb = v_ref[b]
        si = _dot(vb, vb, ((1,), (1,)))
        cur = si
        thr = si
        for _ in range(4):
            m = jnp.max(cur, axis=1, keepdims=True)
            thr = m
            eq = cur == m
            idx = jnp.min(jnp.where(eq, iot, 10 ** 6), axis=1, keepdims=True)
            cur = jnp.where(iot == idx, -1e30, cur)
        adj = (si >= thr).astype(F32)
        deg = jnp.sum(adj, axis=1, keepdims=True)
        dinv = lax.rsqrt(deg)
        Vx = _dot(vb, VW_ref[...], ((1,), (1,))) + Vb_ref[...]
        Ux = _dot(vb, UW_ref[...], ((1,), (1,))) + Ub_ref[...]
        aggb = dinv * _dot(adj, dinv * Vx, ((1,), (0,))) + Ux
        agg_ref[b] = aggb
        msum = msum + jnp.sum(aggb, axis=1, keepdims=True)
    mcol = msum / float(B * C)

    vsum = jnp.zeros((K, 1), F32)
    for b in range(B):
        d = agg_ref[b] - mcol
        vsum = vsum + jnp.sum(d * d, axis=1, keepdims=True)
    inv_col = lax.rsqrt(vsum / float(B * C) + 1e-5)

    ms3 = jnp.zeros((1, C), F32)
    for b in range(B):
        aggn = (agg_ref[b] - mcol) * inv_col * gK_ref[...] + bK_ref[...]
        vgnn = jnp.maximum(v_ref[b] + aggn, 0.0)
        fpb = (_dot(vgnn, pL_ref[...], ((1,), (1,)))
               + _dot(lin[b:b + 1, :], pR_ref[...], ((1,), (1,)))
               + pb_ref[...])
        fp_ref[b] = fpb
        ms3 = ms3 + jnp.sum(fpb, axis=0, keepdims=True)
    m3 = ms3 / float(B * K)

    vs3 = jnp.zeros((1, C), F32)
    for b in range(B):
        d = fp_ref[b] - m3
        vs3 = vs3 + jnp.sum(d * d, axis=0, keepdims=True)
    inv3 = lax.rsqrt(vs3 / float(B * K) + 1e-5)

    scr = jnp.maximum(sc_ref[...], 0.0)
    sn = jnp.sqrt(jnp.sum(scr * scr, axis=1, keepdims=True))
    scn = scr / jnp.maximum(sn, 1e-12)

    rows = []
    for b in range(B):
        y = (fp_ref[b] - m3) * inv3 * pg_ref[...] + pbe_ref[...]
        f = jnp.maximum(y, 0.0)
        fn = jnp.sqrt(jnp.sum(f * f, axis=1, keepdims=True))
        fhat = f / jnp.maximum(fn, 1e-12)
        rows.append(_dot(ones_row, fhat * scn, ((1,), (1,))))
    out_ref[...] = jnp.concatenate(rows, axis=0)


def _stage3(v_bkc, XH, args):
    full = lambda a: pl.BlockSpec(a.shape, lambda: tuple(0 for _ in a.shape))
    ins = [v_bkc, XH] + list(args)
    return pl.pallas_call(
        _s3_body,
        in_specs=[full(a) for a in ins],
        out_specs=pl.BlockSpec((B, K), lambda: (0, 0)),
        out_shape=jax.ShapeDtypeStruct((B, K), F32),
        scratch_shapes=[pltpu.VMEM((B, K, C), F32), pltpu.VMEM((B, K, C), F32)],
    )(*ins)


def kernel(x, params):
    p = params

    def taps(name, kh, kw):
        w = p[name]["w"]
        return w[:, 0].reshape(C, kh * kw).T

    tap_ws = (taps("conv0", 5, 5), taps("conv0_1", 1, 7), taps("conv0_2", 7, 1),
              taps("conv1_1", 1, 11), taps("conv1_2", 11, 1),
              taps("conv2_1", 1, 21), taps("conv2_2", 21, 1))
    cbias = jnp.stack([p["conv0"]["b"], p["conv0_1"]["b"], p["conv0_2"]["b"],
                       p["conv1_1"]["b"], p["conv1_2"]["b"],
                       p["conv2_1"]["b"], p["conv2_2"]["b"],
                       p["conv3"]["b"]], axis=0)
    W3 = p["conv3"]["w"][:, :, 0, 0]

    xs, XH = _stage1(x, tap_ws, cbias, W3)

    v_bkc = _stage2(xs, p["cls_W"], p["cls_b"][:, None, :],
                    p["cls_g"][:, None, :], p["cls_be"][:, None, :])

    row = lambda a: a[None, :]
    args = (p["V_W"], row(p["V_b"]), p["U_W"], row(p["U_b"]),
            jnp.broadcast_to(p["bnv_g"][:, None], (K, C)),
            jnp.broadcast_to(p["bnv_b"][:, None], (K, C)),
            p["c1_w"][:, :, 0, 0], row(p["c1_b"]),
            p["c2_w"][:, :, 0, 0], row(p["c2_b"]),
            p["c3_w"][:, :, 0, 0], row(p["c3_b"]),
            p["c4_w"][:, :, 0, 0], row(p["c4_b"]),
            p["ccl_W"], row(p["ccl_b"]), row(p["ccl_g"]), row(p["ccl_be"]),
            p["post_W"][:, :C], p["post_W"][:, C:], row(p["post_b"]),
            row(p["post_g"]), row(p["post_be"]), p["sc"])
    return _stage3(v_bkc, XH, args)


# speedup vs baseline: 1.2752x; 1.2752x over previous
import jax
import jax.numpy as jnp
from jax import lax
from jax.experimental import pallas as pl
from jax.experimental.pallas import tpu as pltpu

B, HW, C, K = 16, 196, 384, 12
REL = 48
H = W = 14
N = B * HW
PAD = 144
ROWS = PAD + HW + PAD
F32 = jnp.float32


def _dot(a, b, dims):
    return lax.dot_general(a, b, (dims, ((), ())), preferred_element_type=F32)


def _s1_body(x_ref, w0, w01, w02, w11, w12, w21, w22, cb_ref, W3_ref,
             xs_ref, xh_ref, pad_ref, s0_ref, t0_ref, t1_ref):
    b = pl.program_id(0)
    wio = lax.broadcasted_iota(jnp.int32, (HW, 1), 0) % W

    @pl.when(b == 0)
    def _():
        pad_ref[...] = jnp.zeros((ROWS, C), F32)

    def put(img):
        pad_ref[pl.ds(PAD, HW), :] = img

    def conv(wt_ref, kh, kw, ph, pw):
        acc = jnp.zeros((HW, C), F32)
        t = 0
        for i in range(kh):
            for j in range(kw):
                delta = (i - ph) * W + (j - pw)
                sl = pad_ref[pl.ds(PAD + delta, HW), :]
                dw = j - pw
                if dw != 0:
                    m = (wio + dw >= 0) & (wio + dw < W)
                    sl = jnp.where(m, sl, 0.0)
                acc = acc + sl * wt_ref[pl.ds(t, 1), :]
                t += 1
        return acc

    def brow(i):
        return cb_ref[pl.ds(i, 1), :]

    X = x_ref[0]
    put(X)
    s0 = conv(w0, 5, 5, 2, 2) + brow(0)
    s0_ref[...] = s0
    put(s0)
    t0_ref[...] = conv(w01, 1, 7, 0, 3) + brow(1)
    t1_ref[...] = conv(w11, 1, 11, 0, 5) + brow(3)
    aw2 = conv(w21, 1, 21, 0, 10) + brow(5)
    put(aw2)
    acc = s0_ref[...] + conv(w22, 21, 1, 10, 0) + brow(6)
    put(t0_ref[...])
    acc = acc + conv(w02, 7, 1, 3, 0) + brow(2)
    put(t1_ref[...])
    acc = acc + conv(w12, 11, 1, 5, 0) + brow(4)
    attn3 = _dot(acc, W3_ref[...], ((1,), (1,))) + brow(7)
    xs_ref[0] = attn3 * X
    xh = jnp.zeros((W, C), F32)
    for t in range(H):
        xh = xh + X[t * W:(t + 1) * W, :]
    xh_ref[0] = xh


def _stage1(x, taps, cbias, W3):
    w0, w01, w02, w11, w12, w21, w22 = taps
    full = lambda s: pl.BlockSpec(s, lambda b: tuple(0 for _ in s))
    return pl.pallas_call(
        _s1_body,
        grid=(B,),
        in_specs=[pl.BlockSpec((1, HW, C), lambda b: (b, 0, 0))]
        + [full(w.shape) for w in (w0, w01, w02, w11, w12, w21, w22)]
        + [full(cbias.shape), full(W3.shape)],
        out_specs=[pl.BlockSpec((1, HW, C), lambda b: (b, 0, 0)),
                   pl.BlockSpec((1, W, C), lambda b: (b, 0, 0))],
        out_shape=[jax.ShapeDtypeStruct((B, HW, C), F32),
                   jax.ShapeDtypeStruct((B, W, C), F32)],
        scratch_shapes=[pltpu.VMEM((ROWS, C), F32), pltpu.VMEM((HW, C), F32),
                        pltpu.VMEM((HW, C), F32), pltpu.VMEM((HW, C), F32)],
        compiler_params=pltpu.CompilerParams(
            dimension_semantics=("arbitrary",)),
    )(x, w0, w01, w02, w11, w12, w21, w22, cbias, W3)


def _s2_body(xs_ref, wcat_ref, cb_ref, cg_ref, cbe_ref, out_ref, ab_ref):
    b = pl.program_id(0)

    @pl.when(b == 0)
    def _():
        ssum = jnp.zeros((1, C), F32)
        for bb in range(B):
            ssum = ssum + jnp.sum(xs_ref[bb], axis=0, keepdims=True)
        s = ssum / N
        G = jnp.zeros((C, C), F32)
        for bb in range(B):
            xc = xs_ref[bb] - s
            G = G + _dot(xc, xc, ((0,), (0,)))
        ones = jnp.ones((1, C), F32)
        for k in range(K):
            Ws = wcat_ref[:, k * C:(k + 1) * C]
            Ez = _dot(s, Ws, ((1,), (0,)))
            GW = _dot(G, Ws, ((1,), (0,)))
            var = _dot(ones, Ws * GW, ((1,), (0,))) / N
            alpha = cg_ref[:, k * C:(k + 1) * C] / jnp.sqrt(var + 1e-5)
            beta = cbe_ref[:, k * C:(k + 1) * C] - Ez * alpha
            ab_ref[pl.ds(0, 1), k * C:(k + 1) * C] = alpha
            ab_ref[pl.ds(1, 1), k * C:(k + 1) * C] = beta

    Z = _dot(xs_ref[b], wcat_ref[...], ((1,), (0,)))
    y = jnp.maximum(Z * ab_ref[pl.ds(0, 1), :] + ab_ref[pl.ds(1, 1), :], 0.0)
    ones_n = jnp.ones((1, HW), F32)
    out_ref[...] = _dot(ones_n, y, ((1,), (0,)))[None] / HW


def _stage2(xs, cW, cb, cg, cbe):
    wcat = cW.transpose(2, 0, 1).reshape(C, K * C)
    cbf = cb.reshape(1, K * C)
    cgf = cg.reshape(1, K * C)
    cbef = cbe.reshape(1, K * C)
    full = lambda s: pl.BlockSpec(s, lambda b: tuple(0 for _ in s))
    out = pl.pallas_call(
        _s2_body,
        grid=(B,),
        in_specs=[full((B, HW, C)), full((C, K * C)), full((1, K * C)),
                  full((1, K * C)), full((1, K * C))],
        out_specs=pl.BlockSpec((1, 1, K * C), lambda b: (b, 0, 0)),
        out_shape=jax.ShapeDtypeStruct((B, 1, K * C), F32),
        scratch_shapes=[pltpu.VMEM((2, K * C), F32)],
        compiler_params=pltpu.CompilerParams(
            dimension_semantics=("arbitrary",)),
    )(xs, wcat, cbf, cgf, cbef)
    return out.reshape(B, K, C)


def _s3_body(v_ref, xh_ref, VW_ref, Vb_ref, UW_ref, Ub_ref, gK_ref, bK_ref,
             c1w_ref, c1b_ref, c2w_ref, c2b_ref, c3w_ref, c3b_ref,
             c4w_ref, c4b_ref, cclW_ref, cclb_ref, cclg_ref, cclbe_ref,
             pL_ref, pR_ref, pb_ref, pg_ref, pbe_ref, sc_ref,
             out_ref, agg_ref, fp_ref):
    ones_row = jnp.ones((1, C), F32)

    pool_rows = []
    for b in range(B):
        XHb = xh_ref[b]
        XHm = XHb * (1.0 / W)
        x1 = _dot(XHm, c1w_ref[...], ((1,), (1,))) + c1b_ref[...]
        x2 = _dot(XHm, c2w_ref[...], ((1,), (1,))) + c2b_ref[...]
        TT = jnp.zeros((W, REL), F32)
        for u in range(W):
            TT = TT + jnp.tanh(x1[u:u + 1, :] - x2)
        P1T = _dot(TT, c4w_ref[...], ((1,), (1,))) + float(W) * c4b_ref[...]
        S3T = _dot(XHb, c3w_ref[...], ((1,), (1,))) + float(W) * c3b_ref[...]
        pool_rows.append(jnp.sum(P1T * S3T, axis=0, keepdims=True) / float(HW))
    pool = jnp.concatenate(pool_rows, axis=0)

    y1 = _dot(pool, cclW_ref[...], ((1,), (1,))) + cclb_ref[...]
    m2 = jnp.mean(y1, axis=0, keepdims=True)
    v2 = jnp.mean((y1 - m2) ** 2, axis=0, keepdims=True)
    lin = jnp.maximum(
        (y1 - m2) * lax.rsqrt(v2 + 1e-5) * cclg_ref[...] + cclbe_ref[...], 0.0)

    msum = jnp.zeros((K, 1), F32)
    iot = lax.broadcasted_iota(jnp.int32, (K, K), 1)
    for b in range(B):
        vb = v_ref[b]
        si = _dot(vb, vb, ((1,), (1,)))
        cur = si
        thr = si
        for _ in range(4):
            m = jnp.max(cur, axis=1, keepdims=True)
            thr = m
            eq = cur == m
            idx = jnp.min(jnp.where(eq, iot, 10 ** 6), axis=1, keepdims=True)
            cur = jnp.where(iot == idx, -1e30, cur)
        adj = (si >= thr).astype(F32)
        deg = jnp.sum(adj, axis=1, keepdims=True)
        dinv = lax.rsqrt(deg)
        Vx = _dot(vb, VW_ref[...], ((1,), (1,))) + Vb_ref[...]
        Ux = _dot(vb, UW_ref[...], ((1,), (1,))) + Ub_ref[...]
        aggb = dinv * _dot(adj, dinv * Vx, ((1,), (0,))) + Ux
        agg_ref[b] = aggb
        msum = msum + jnp.sum(aggb, axis=1, keepdims=True)
    mcol = msum / float(B * C)

    vsum = jnp.zeros((K, 1), F32)
    for b in range(B):
        d = agg_ref[b] - mcol
        vsum = vsum + jnp.sum(d * d, axis=1, keepdims=True)
    inv_col = lax.rsqrt(vsum / float(B * C) + 1e-5)

    ms3 = jnp.zeros((1, C), F32)
    for b in range(B):
        aggn = (agg_ref[b] - mcol) * inv_col * gK_ref[...] + bK_ref[...]
        vgnn = jnp.maximum(v_ref[b] + aggn, 0.0)
        fpb = (_dot(vgnn, pL_ref[...], ((1,), (1,)))
               + _dot(lin[b:b + 1, :], pR_ref[...], ((1,), (1,)))
               + pb_ref[...])
        fp_ref[b] = fpb
        ms3 = ms3 + jnp.sum(fpb, axis=0, keepdims=True)
    m3 = ms3 / float(B * K)

    vs3 = jnp.zeros((1, C), F32)
    for b in range(B):
        d = fp_ref[b] - m3
        vs3 = vs3 + jnp.sum(d * d, axis=0, keepdims=True)
    inv3 = lax.rsqrt(vs3 / float(B * K) + 1e-5)

    scr = jnp.maximum(sc_ref[...], 0.0)
    sn = jnp.sqrt(jnp.sum(scr * scr, axis=1, keepdims=True))
    scn = scr / jnp.maximum(sn, 1e-12)

    rows = []
    for b in range(B):
        y = (fp_ref[b] - m3) * inv3 * pg_ref[...] + pbe_ref[...]
        f = jnp.maximum(y, 0.0)
        fn = jnp.sqrt(jnp.sum(f * f, axis=1, keepdims=True))
        fhat = f / jnp.maximum(fn, 1e-12)
        rows.append(_dot(ones_row, fhat * scn, ((1,), (1,))))
    out_ref[...] = jnp.concatenate(rows, axis=0)


def _stage3(v_bkc, XH, args):
    full = lambda a: pl.BlockSpec(a.shape, lambda: tuple(0 for _ in a.shape))
    ins = [v_bkc, XH] + list(args)
    return pl.pallas_call(
        _s3_body,
        in_specs=[full(a) for a in ins],
        out_specs=pl.BlockSpec((B, K), lambda: (0, 0)),
        out_shape=jax.ShapeDtypeStruct((B, K), F32),
        scratch_shapes=[pltpu.VMEM((B, K, C), F32), pltpu.VMEM((B, K, C), F32)],
    )(*ins)


def kernel(x, params):
    p = params

    def taps(name, kh, kw):
        w = p[name]["w"]
        return w[:, 0].reshape(C, kh * kw).T

    tap_ws = (taps("conv0", 5, 5), taps("conv0_1", 1, 7), taps("conv0_2", 7, 1),
              taps("conv1_1", 1, 11), taps("conv1_2", 11, 1),
              taps("conv2_1", 1, 21), taps("conv2_2", 21, 1))
    cbias = jnp.stack([p["conv0"]["b"], p["conv0_1"]["b"], p["conv0_2"]["b"],
                       p["conv1_1"]["b"], p["conv1_2"]["b"],
                       p["conv2_1"]["b"], p["conv2_2"]["b"],
                       p["conv3"]["b"]], axis=0)
    W3 = p["conv3"]["w"][:, :, 0, 0]

    xs, XH = _stage1(x, tap_ws, cbias, W3)

    v_bkc = _stage2(xs, p["cls_W"], p["cls_b"], p["cls_g"], p["cls_be"])

    row = lambda a: a[None, :]
    args = (p["V_W"], row(p["V_b"]), p["U_W"], row(p["U_b"]),
            jnp.broadcast_to(p["bnv_g"][:, None], (K, C)),
            jnp.broadcast_to(p["bnv_b"][:, None], (K, C)),
            p["c1_w"][:, :, 0, 0], row(p["c1_b"]),
            p["c2_w"][:, :, 0, 0], row(p["c2_b"]),
            p["c3_w"][:, :, 0, 0], row(p["c3_b"]),
            p["c4_w"][:, :, 0, 0], row(p["c4_b"]),
            p["ccl_W"], row(p["ccl_b"]), row(p["ccl_g"]), row(p["ccl_be"]),
            p["post_W"][:, :C], p["post_W"][:, C:], row(p["post_b"]),
            row(p["post_g"]), row(p["post_be"]), p["sc"])
    return _stage3(v_bkc, XH, args)


# speedup vs baseline: 1.3747x; 1.0780x over previous
import jax
import jax.numpy as jnp
from jax import lax
from jax.experimental import pallas as pl
from jax.experimental.pallas import tpu as pltpu

B, HW, C, K = 16, 196, 384, 12
REL = 48
H = W = 14
N = B * HW
PAD = 144
ROWS = PAD + HW + PAD
F32 = jnp.float32


def _dot(a, b, dims):
    return lax.dot_general(a, b, (dims, ((), ())), preferred_element_type=F32)


def _s1_body(x_ref, w0, w01, w02, w11, w12, w21, w22, cb_ref, W3_ref,
             xs_ref, xh_ref, pad_ref, s0_ref, t0_ref, t1_ref):
    b = pl.program_id(0)
    wio = lax.broadcasted_iota(jnp.int32, (HW, 1), 0) % W

    @pl.when(b == 0)
    def _():
        pad_ref[...] = jnp.zeros((ROWS, C), F32)

    def put(img):
        pad_ref[pl.ds(PAD, HW), :] = img

    def conv(wt_ref, kh, kw, ph, pw):
        acc = jnp.zeros((HW, C), F32)
        t = 0
        for i in range(kh):
            for j in range(kw):
                delta = (i - ph) * W + (j - pw)
                sl = pad_ref[pl.ds(PAD + delta, HW), :]
                dw = j - pw
                if dw != 0:
                    m = (wio + dw >= 0) & (wio + dw < W)
                    sl = jnp.where(m, sl, 0.0)
                acc = acc + sl * wt_ref[pl.ds(t, 1), :]
                t += 1
        return acc

    def brow(i):
        return cb_ref[pl.ds(i, 1), :]

    X = x_ref[0]
    put(X)
    s0 = conv(w0, 5, 5, 2, 2) + brow(0)
    s0_ref[...] = s0
    put(s0)
    a0 = jnp.zeros((HW, C), F32)
    a1 = jnp.zeros((HW, C), F32)
    a2 = jnp.zeros((HW, C), F32)
    for dw in range(-10, 11):
        sl = pad_ref[pl.ds(PAD + dw, HW), :]
        m = (wio + dw >= 0) & (wio + dw < W)
        sl = jnp.where(m, sl, 0.0)
        a2 = a2 + sl * w21[pl.ds(dw + 10, 1), :]
        if abs(dw) <= 5:
            a1 = a1 + sl * w11[pl.ds(dw + 5, 1), :]
        if abs(dw) <= 3:
            a0 = a0 + sl * w01[pl.ds(dw + 3, 1), :]
    t0_ref[...] = a0 + brow(1)
    t1_ref[...] = a1 + brow(3)
    aw2 = a2 + brow(5)
    put(aw2)
    acc = s0_ref[...] + conv(w22, 21, 1, 10, 0) + brow(6)
    put(t0_ref[...])
    acc = acc + conv(w02, 7, 1, 3, 0) + brow(2)
    put(t1_ref[...])
    acc = acc + conv(w12, 11, 1, 5, 0) + brow(4)
    attn3 = _dot(acc, W3_ref[...], ((1,), (1,))) + brow(7)
    xs_ref[0] = attn3 * X
    xh = jnp.zeros((W, C), F32)
    for t in range(H):
        xh = xh + X[t * W:(t + 1) * W, :]
    xh_ref[0] = xh


def _stage1(x, taps, cbias, W3):
    w0, w01, w02, w11, w12, w21, w22 = taps
    full = lambda s: pl.BlockSpec(s, lambda b: tuple(0 for _ in s))
    return pl.pallas_call(
        _s1_body,
        grid=(B,),
        in_specs=[pl.BlockSpec((1, HW, C), lambda b: (b, 0, 0))]
        + [full(w.shape) for w in (w0, w01, w02, w11, w12, w21, w22)]
        + [full(cbias.shape), full(W3.shape)],
        out_specs=[pl.BlockSpec((1, HW, C), lambda b: (b, 0, 0)),
                   pl.BlockSpec((1, W, C), lambda b: (b, 0, 0))],
        out_shape=[jax.ShapeDtypeStruct((B, HW, C), F32),
                   jax.ShapeDtypeStruct((B, W, C), F32)],
        scratch_shapes=[pltpu.VMEM((ROWS, C), F32), pltpu.VMEM((HW, C), F32),
                        pltpu.VMEM((HW, C), F32), pltpu.VMEM((HW, C), F32)],
        compiler_params=pltpu.CompilerParams(
            dimension_semantics=("arbitrary",)),
    )(x, w0, w01, w02, w11, w12, w21, w22, cbias, W3)


def _s2_body(xs_ref, wcat_ref, cb_ref, cg_ref, cbe_ref, out_ref, ab_ref):
    b = pl.program_id(0)

    @pl.when(b == 0)
    def _():
        ssum = jnp.zeros((1, C), F32)
        for bb in range(B):
            ssum = ssum + jnp.sum(xs_ref[bb], axis=0, keepdims=True)
        s = ssum / N
        G = jnp.zeros((C, C), F32)
        for bb in range(B):
            xc = xs_ref[bb] - s
            G = G + _dot(xc, xc, ((0,), (0,)))
        ones = jnp.ones((1, C), F32)
        for k in range(K):
            Ws = wcat_ref[k * C:(k + 1) * C, :]
            Ez = _dot(s, Ws, ((1,), (1,)))
            WG = _dot(Ws, G, ((1,), (0,)))
            var = _dot(ones, WG * Ws, ((1,), (1,))) / N
            alpha = cg_ref[:, k * C:(k + 1) * C] / jnp.sqrt(var + 1e-5)
            beta = cbe_ref[:, k * C:(k + 1) * C] - Ez * alpha
            ab_ref[pl.ds(0, 1), k * C:(k + 1) * C] = alpha
            ab_ref[pl.ds(1, 1), k * C:(k + 1) * C] = beta

    Z = _dot(xs_ref[b], wcat_ref[...], ((1,), (1,)))
    y = jnp.maximum(Z * ab_ref[pl.ds(0, 1), :] + ab_ref[pl.ds(1, 1), :], 0.0)
    ones_n = jnp.ones((1, HW), F32)
    out_ref[...] = _dot(ones_n, y, ((1,), (0,)))[None] / HW


def _stage2(xs, cW, cb, cg, cbe):
    wcat = cW.reshape(K * C, C)
    cbf = cb.reshape(1, K * C)
    cgf = cg.reshape(1, K * C)
    cbef = cbe.reshape(1, K * C)
    full = lambda s: pl.BlockSpec(s, lambda b: tuple(0 for _ in s))
    out = pl.pallas_call(
        _s2_body,
        grid=(B,),
        in_specs=[full((B, HW, C)), full((K * C, C)), full((1, K * C)),
                  full((1, K * C)), full((1, K * C))],
        out_specs=pl.BlockSpec((1, 1, K * C), lambda b: (b, 0, 0)),
        out_shape=jax.ShapeDtypeStruct((B, 1, K * C), F32),
        scratch_shapes=[pltpu.VMEM((2, K * C), F32)],
        compiler_params=pltpu.CompilerParams(
            dimension_semantics=("arbitrary",)),
    )(xs, wcat, cbf, cgf, cbef)
    return out.reshape(B, K, C)


def _s3_body(v_ref, xh_ref, VW_ref, Vb_ref, UW_ref, Ub_ref, gK_ref, bK_ref,
             c1w_ref, c1b_ref, c2w_ref, c2b_ref, c3w_ref, c3b_ref,
             c4w_ref, c4b_ref, cclW_ref, cclb_ref, cclg_ref, cclbe_ref,
             pW_ref, pb_ref, pg_ref, pbe_ref, sc_ref,
             out_ref, agg_ref, fp_ref):
    ones_row = jnp.ones((1, C), F32)
    pL = pW_ref[:, 0:C]
    pR = pW_ref[:, C:2 * C]

    pool_rows = []
    for b in range(B):
        XHb = xh_ref[b]
        XHm = XHb * (1.0 / W)
        x1 = _dot(XHm, c1w_ref[...], ((1,), (1,))) + c1b_ref[...]
        x2 = _dot(XHm, c2w_ref[...], ((1,), (1,))) + c2b_ref[...]
        TT = jnp.zeros((W, REL), F32)
        for u in range(W):
            TT = TT + jnp.tanh(x1[u:u + 1, :] - x2)
        P1T = _dot(TT, c4w_ref[...], ((1,), (1,))) + float(W) * c4b_ref[...]
        S3T = _dot(XHb, c3w_ref[...], ((1,), (1,))) + float(W) * c3b_ref[...]
        pool_rows.append(jnp.sum(P1T * S3T, axis=0, keepdims=True) / float(HW))
    pool = jnp.concatenate(pool_rows, axis=0)

    y1 = _dot(pool, cclW_ref[...], ((1,), (1,))) + cclb_ref[...]
    m2 = jnp.mean(y1, axis=0, keepdims=True)
    v2 = jnp.mean((y1 - m2) ** 2, axis=0, keepdims=True)
    lin = jnp.maximum(
        (y1 - m2) * lax.rsqrt(v2 + 1e-5) * cclg_ref[...] + cclbe_ref[...], 0.0)

    msum = jnp.zeros((K, 1), F32)
    iot = lax.broadcasted_iota(jnp.int32, (K, K), 1)
    for b in range(B):
        vb = v_ref[b]
        si = _dot(vb, vb, ((1,), (1,)))
        cur = si
        thr = si
        for _ in range(4):
            m = jnp.max(cur, axis=1, keepdims=True)
            thr = m
            eq = cur == m
            idx = jnp.min(jnp.where(eq, iot, 10 ** 6), axis=1, keepdims=True)
            cur = jnp.where(iot == idx, -1e30, cur)
        adj = (si >= thr).astype(F32)
        deg = jnp.sum(adj, axis=1, keepdims=True)
        dinv = lax.rsqrt(deg)
        Vx = _dot(vb, VW_ref[...], ((1,), (1,))) + Vb_ref[...]
        Ux = _dot(vb, UW_ref[...], ((1,), (1,))) + Ub_ref[...]
        aggb = dinv * _dot(adj, dinv * Vx, ((1,), (0,))) + Ux
        agg_ref[b] = aggb
        msum = msum + jnp.sum(aggb, axis=1, keepdims=True)
    mcol = msum / float(B * C)

    vsum = jnp.zeros((K, 1), F32)
    for b in range(B):
        d = agg_ref[b] - mcol
        vsum = vsum + jnp.sum(d * d, axis=1, keepdims=True)
    inv_col = lax.rsqrt(vsum / float(B * C) + 1e-5)

    ms3 = jnp.zeros((1, C), F32)
    for b in range(B):
        aggn = (agg_ref[b] - mcol) * inv_col * gK_ref[...] + bK_ref[...]
        vgnn = jnp.maximum(v_ref[b] + aggn, 0.0)
        fpb = (_dot(vgnn, pL, ((1,), (1,)))
               + _dot(lin[b:b + 1, :], pR, ((1,), (1,)))
               + pb_ref[...])
        fp_ref[b] = fpb
        ms3 = ms3 + jnp.sum(fpb, axis=0, keepdims=True)
    m3 = ms3 / float(B * K)

    vs3 = jnp.zeros((1, C), F32)
    for b in range(B):
        d = fp_ref[b] - m3
        vs3 = vs3 + jnp.sum(d * d, axis=0, keepdims=True)
    inv3 = lax.rsqrt(vs3 / float(B * K) + 1e-5)

    scr = jnp.maximum(sc_ref[...], 0.0)
    sn = jnp.sqrt(jnp.sum(scr * scr, axis=1, keepdims=True))
    scn = scr / jnp.maximum(sn, 1e-12)

    rows = []
    for b in range(B):
        y = (fp_ref[b] - m3) * inv3 * pg_ref[...] + pbe_ref[...]
        f = jnp.maximum(y, 0.0)
        fn = jnp.sqrt(jnp.sum(f * f, axis=1, keepdims=True))
        fhat = f / jnp.maximum(fn, 1e-12)
        rows.append(_dot(ones_row, fhat * scn, ((1,), (1,))))
    out_ref[...] = jnp.concatenate(rows, axis=0)


def _stage3(v_bkc, XH, args):
    full = lambda a: pl.BlockSpec(a.shape, lambda: tuple(0 for _ in a.shape))
    ins = [v_bkc, XH] + list(args)
    return pl.pallas_call(
        _s3_body,
        in_specs=[full(a) for a in ins],
        out_specs=pl.BlockSpec((B, K), lambda: (0, 0)),
        out_shape=jax.ShapeDtypeStruct((B, K), F32),
        scratch_shapes=[pltpu.VMEM((B, K, C), F32), pltpu.VMEM((B, K, C), F32)],
    )(*ins)


def kernel(x, params):
    p = params

    def taps(name, kh, kw):
        w = p[name]["w"]
        return w[:, 0].reshape(C, kh * kw).T

    tap_ws = (taps("conv0", 5, 5), taps("conv0_1", 1, 7), taps("conv0_2", 7, 1),
              taps("conv1_1", 1, 11), taps("conv1_2", 11, 1),
              taps("conv2_1", 1, 21), taps("conv2_2", 21, 1))
    cbias = jnp.stack([p["conv0"]["b"], p["conv0_1"]["b"], p["conv0_2"]["b"],
                       p["conv1_1"]["b"], p["conv1_2"]["b"],
                       p["conv2_1"]["b"], p["conv2_2"]["b"],
                       p["conv3"]["b"]], axis=0)
    W3 = p["conv3"]["w"][:, :, 0, 0]

    xs, XH = _stage1(x, tap_ws, cbias, W3)

    v_bkc = _stage2(xs, p["cls_W"], p["cls_b"], p["cls_g"], p["cls_be"])

    row = lambda a: a[None, :]
    args = (p["V_W"], row(p["V_b"]), p["U_W"], row(p["U_b"]),
            jnp.broadcast_to(p["bnv_g"][:, None], (K, C)),
            jnp.broadcast_to(p["bnv_b"][:, None], (K, C)),
            p["c1_w"][:, :, 0, 0], row(p["c1_b"]),
            p["c2_w"][:, :, 0, 0], row(p["c2_b"]),
            p["c3_w"][:, :, 0, 0], row(p["c3_b"]),
            p["c4_w"][:, :, 0, 0], row(p["c4_b"]),
            p["ccl_W"], row(p["ccl_b"]), row(p["ccl_g"]), row(p["ccl_be"]),
            p["post_W"], row(p["post_b"]),
            row(p["post_g"]), row(p["post_be"]), p["sc"])
    return _stage3(v_bkc, XH, args)
